# baseline (speedup 1.0000x reference)
import jax
import jax.numpy as jnp
from jax import lax
from jax.experimental import pallas as pl
from jax.experimental.pallas import tpu as pltpu

DIM = 768
E = 8
TOPK = 2
NSH = 2
L = 2048
H = 4 * DIM
HSH = 2 * DIM

T = 256
NB = 24
NP = NB * T

HB = 2
HC = H // HB

TS = 256
CC = 2048

BF = jnp.bfloat16


def _routed_block_kernel(bexp_ref, rows_ref, gv_ref, x_ref, w1a_ref, w1g_ref,
                         w2_ref, wp_ref, ys_ref, y_s, w1a_s, w1g_s, w2_s):
    j = pl.program_id(0)
    b = pl.program_id(1)

    new_w = (b == 0) | (bexp_ref[b] != bexp_ref[jnp.maximum(b - 1, 0)])

    @pl.when(new_w)
    def _():
        w1a_s[...] = w1a_ref[0].astype(BF)
        w1g_s[...] = w1g_ref[0].astype(BF)
        w2_s[...] = w2_ref[0].astype(BF)

    rows = rows_ref[0][:, 0:1]
    iota = lax.broadcasted_iota(jnp.int32, (T, L), 1)
    oh = (rows == iota).astype(BF)
    xg = jnp.dot(oh, x_ref[...],
                 preferred_element_type=jnp.float32).astype(BF)

    a = jnp.dot(xg, w1a_s[...], preferred_element_type=jnp.float32)
    g = jnp.dot(xg, w1g_s[...], preferred_element_type=jnp.float32)
    u = (a * jax.nn.sigmoid(a) * g).astype(BF)

    part = jnp.dot(u, w2_s[...], preferred_element_type=jnp.float32)

    @pl.when(j == 0)
    def _():
        y_s[b] = part.astype(BF)

    @pl.when((j > 0) & (j < HB - 1))
    def _():
        y_s[b] = (y_s[b].astype(jnp.float32) + part).astype(BF)

    @pl.when(j == HB - 1)
    def _():
        y = y_s[b].astype(jnp.float32) + part
        z = jnp.dot(y.astype(BF), wp_ref[0], preferred_element_type=jnp.float32)
        gv = gv_ref[0][:, 0:1]
        ys_ref[...] = (z * gv).astype(BF)


def _combine_shared_kernel(x_ref, w1_ref, w2_ref, ys_ref, d0_ref, d1_ref,
                           out_ref):
    d0 = d0_ref[0][:, 0:1]
    d1 = d1_ref[0][:, 0:1]

    acc = jnp.zeros((TS, DIM), jnp.float32)
    for c in range(NP // CC):
        iota = lax.broadcasted_iota(jnp.int32, (TS, CC), 1) + c * CC
        oh2 = (d0 == iota).astype(BF) + (d1 == iota).astype(BF)
        acc = acc + jnp.dot(oh2, ys_ref[c * CC:(c + 1) * CC, :],
                            preferred_element_type=jnp.float32)

    xt = x_ref[...]
    for s in range(NSH):
        h = jnp.dot(xt, w1_ref[s], preferred_element_type=jnp.float32)
        a = h[:, :HSH]
        g = h[:, HSH:]
        u = (a * jax.nn.sigmoid(a) * g).astype(BF)
        y = jnp.dot(u, w2_ref[s], preferred_element_type=jnp.float32)
        acc = acc + (1.0 / NSH) * y
    out_ref[...] = acc


@jax.jit
def kernel(x, gate_w, exp_w1, exp_w2, exp_wp, sh_w1, sh_w2):
    xf = x[0]
    xb = xf.astype(BF)

    gate_logits = x @ gate_w
    top_k_logits, top_k_idx = lax.top_k(gate_logits, TOPK)
    top_k_gates = jax.nn.softmax(top_k_logits, axis=-1)

    expert_mask = jax.nn.one_hot(top_k_idx[:, :, 0], E, dtype=jnp.float32)
    counts0 = expert_mask.sum(axis=(0, 1))
    counts0 = counts0 / counts0.sum()
    target = jnp.ones_like(counts0) / E
    lb_loss = jnp.mean((counts0 - target) ** 2)

    e_flat = top_k_idx.reshape(-1)
    g_flat = top_k_gates.reshape(-1).astype(jnp.float32)
    tok = jnp.arange(TOPK * L, dtype=jnp.int32) // TOPK

    onehot = (e_flat[:, None] == jnp.arange(E)[None, :]).astype(jnp.float32)
    cum = jnp.cumsum(onehot, axis=0)
    rank = ((cum - onehot) * onehot).sum(axis=1)
    counts = cum[-1]
    blocks_pe = jnp.floor((counts + T - 1) / T)
    bcum = jnp.cumsum(blocks_pe)
    bfirst = bcum - blocks_pe
    bfirst_pa = (bfirst[None, :] * onehot).sum(axis=1)
    dest = (bfirst_pa * T + rank).astype(jnp.int32)

    rows_flat = jnp.zeros((NP,), jnp.int32).at[dest].set(tok)
    gv_flat = jnp.zeros((NP,), jnp.float32).at[dest].set(g_flat)
    bexp = jnp.sum(jnp.arange(NB, dtype=jnp.float32)[:, None] >= bcum[None, :],
                   axis=1).astype(jnp.int32)
    bexp = jnp.minimum(bexp, E - 1)

    rows_bc = jnp.broadcast_to(rows_flat.reshape(NB, T, 1), (NB, T, 128))
    gv_bc = jnp.broadcast_to(gv_flat.reshape(NB, T, 1), (NB, T, 128))

    dest2 = dest.reshape(L, TOPK)
    d0_bc = jnp.broadcast_to(dest2[:, 0].reshape(L // TS, TS, 1),
                             (L // TS, TS, 128))
    d1_bc = jnp.broadcast_to(dest2[:, 1].reshape(L // TS, TS, 1),
                             (L // TS, TS, 128))

    wpb = exp_wp.astype(BF)

    ys = pl.pallas_call(
        _routed_block_kernel,
        grid_spec=pltpu.PrefetchScalarGridSpec(
            num_scalar_prefetch=1,
            grid=(HB, NB),
            in_specs=[
                pl.BlockSpec((1, T, 128), lambda j, b, bexp: (b, 0, 0)),
                pl.BlockSpec((1, T, 128), lambda j, b, bexp: (b, 0, 0)),
                pl.BlockSpec((L, DIM), lambda j, b, bexp: (0, 0)),
                pl.BlockSpec((1, DIM, HC), lambda j, b, bexp: (bexp[b], 0, j)),
                pl.BlockSpec((1, DIM, HC), lambda j, b, bexp: (bexp[b], 0, j + HB)),
                pl.BlockSpec((1, HC, DIM), lambda j, b, bexp: (bexp[b], j, 0)),
                pl.BlockSpec((1, DIM, DIM), lambda j, b, bexp: (bexp[b], 0, 0)),
            ],
            out_specs=pl.BlockSpec(
                (T, DIM),
                lambda j, b, bexp: (jnp.where(j == HB - 1, b, 0), 0)),
            scratch_shapes=[pltpu.VMEM((NB, T, DIM), BF),
                            pltpu.VMEM((DIM, HC), BF),
                            pltpu.VMEM((DIM, HC), BF),
                            pltpu.VMEM((HC, DIM), BF)],
        ),
        out_shape=jax.ShapeDtypeStruct((NP, DIM), BF),
    )(bexp, rows_bc, gv_bc, xb, exp_w1, exp_w1, exp_w2, wpb)

    out = pl.pallas_call(
        _combine_shared_kernel,
        grid=(L // TS,),
        in_specs=[
            pl.BlockSpec((TS, DIM), lambda i: (i, 0)),
            pl.BlockSpec((NSH, DIM, 2 * HSH), lambda i: (0, 0, 0)),
            pl.BlockSpec((NSH, HSH, DIM), lambda i: (0, 0, 0)),
            pl.BlockSpec((NP, DIM), lambda i: (0, 0)),
            pl.BlockSpec((1, TS, 128), lambda i: (i, 0, 0)),
            pl.BlockSpec((1, TS, 128), lambda i: (i, 0, 0)),
        ],
        out_specs=pl.BlockSpec((TS, DIM), lambda i: (i, 0)),
        out_shape=jax.ShapeDtypeStruct((L, DIM), jnp.float32),
    )(xb, sh_w1.astype(BF), sh_w2.astype(BF), ys, d0_bc, d1_bc)

    return out.reshape(1, L, DIM), lb_loss


# speedup vs baseline: 1.0194x; 1.0194x over previous
import jax
import jax.numpy as jnp
from jax import lax
from jax.experimental import pallas as pl
from jax.experimental.pallas import tpu as pltpu

DIM = 768
E = 8
TOPK = 2
NSH = 2
L = 2048
H = 4 * DIM
HSH = 2 * DIM

T = 128
NB = 40
NP = NB * T

HB = 2
HC = H // HB

TS = 256
CC = 2048

BF = jnp.bfloat16


def _routed_block_kernel(bexp_ref, rows_ref, gv_ref, x_ref, w1a_ref, w1g_ref,
                         w2_ref, wp_ref, ys_ref, y_s):
    j = pl.program_id(0)
    b = pl.program_id(1)

    rows = rows_ref[0][:, 0:1]
    iota = lax.broadcasted_iota(jnp.int32, (T, L), 1)
    oh = (rows == iota).astype(BF)
    xg = jnp.dot(oh, x_ref[...],
                 preferred_element_type=jnp.float32).astype(BF)

    a = jnp.dot(xg, w1a_ref[0].astype(BF), preferred_element_type=jnp.float32)
    g = jnp.dot(xg, w1g_ref[0].astype(BF), preferred_element_type=jnp.float32)
    u = (a * jax.nn.sigmoid(a) * g).astype(BF)

    part = jnp.dot(u, w2_ref[0].astype(BF), preferred_element_type=jnp.float32)

    @pl.when(j == 0)
    def _():
        y_s[b] = part.astype(BF)

    @pl.when((j > 0) & (j < HB - 1))
    def _():
        y_s[b] = (y_s[b].astype(jnp.float32) + part).astype(BF)

    @pl.when(j == HB - 1)
    def _():
        y = y_s[b].astype(jnp.float32) + part
        z = jnp.dot(y.astype(BF), wp_ref[0], preferred_element_type=jnp.float32)
        gv = gv_ref[0][:, 0:1]
        ys_ref[...] = (z * gv).astype(BF)


def _combine_shared_kernel(x_ref, w1_ref, w2_ref, ys_ref, d0_ref, d1_ref,
                           out_ref):
    d0 = d0_ref[0][:, 0:1]
    d1 = d1_ref[0][:, 0:1]

    acc = jnp.zeros((TS, DIM), jnp.float32)
    for c in range(NP // CC):
        iota = lax.broadcasted_iota(jnp.int32, (TS, CC), 1) + c * CC
        oh2 = (d0 == iota).astype(BF) + (d1 == iota).astype(BF)
        acc = acc + jnp.dot(oh2, ys_ref[c * CC:(c + 1) * CC, :],
                            preferred_element_type=jnp.float32)

    xt = x_ref[...]
    for s in range(NSH):
        h = jnp.dot(xt, w1_ref[s], preferred_element_type=jnp.float32)
        a = h[:, :HSH]
        g = h[:, HSH:]
        u = (a * jax.nn.sigmoid(a) * g).astype(BF)
        y = jnp.dot(u, w2_ref[s], preferred_element_type=jnp.float32)
        acc = acc + (1.0 / NSH) * y
    out_ref[...] = acc


@jax.jit
def kernel(x, gate_w, exp_w1, exp_w2, exp_wp, sh_w1, sh_w2):
    xf = x[0]
    xb = xf.astype(BF)

    gate_logits = x @ gate_w
    top_k_logits, top_k_idx = lax.top_k(gate_logits, TOPK)
    top_k_gates = jax.nn.softmax(top_k_logits, axis=-1)

    expert_mask = jax.nn.one_hot(top_k_idx[:, :, 0], E, dtype=jnp.float32)
    counts0 = expert_mask.sum(axis=(0, 1))
    counts0 = counts0 / counts0.sum()
    target = jnp.ones_like(counts0) / E
    lb_loss = jnp.mean((counts0 - target) ** 2)

    e_flat = top_k_idx.reshape(-1)
    g_flat = top_k_gates.reshape(-1).astype(jnp.float32)
    tok = jnp.arange(TOPK * L, dtype=jnp.int32) // TOPK

    onehot = (e_flat[:, None] == jnp.arange(E)[None, :]).astype(jnp.float32)
    cum = jnp.cumsum(onehot, axis=0)
    rank = ((cum - onehot) * onehot).sum(axis=1)
    counts = cum[-1]
    blocks_pe = jnp.floor((counts + T - 1) / T)
    bcum = jnp.cumsum(blocks_pe)
    bfirst = bcum - blocks_pe
    bfirst_pa = (bfirst[None, :] * onehot).sum(axis=1)
    dest = (bfirst_pa * T + rank).astype(jnp.int32)

    rows_flat = jnp.zeros((NP,), jnp.int32).at[dest].set(tok)
    gv_flat = jnp.zeros((NP,), jnp.float32).at[dest].set(g_flat)
    bexp = jnp.sum(jnp.arange(NB, dtype=jnp.float32)[:, None] >= bcum[None, :],
                   axis=1).astype(jnp.int32)
    bexp = jnp.minimum(bexp, E - 1)

    rows_bc = jnp.broadcast_to(rows_flat.reshape(NB, T, 1), (NB, T, 128))
    gv_bc = jnp.broadcast_to(gv_flat.reshape(NB, T, 1), (NB, T, 128))

    dest2 = dest.reshape(L, TOPK)
    d0_bc = jnp.broadcast_to(dest2[:, 0].reshape(L // TS, TS, 1),
                             (L // TS, TS, 128))
    d1_bc = jnp.broadcast_to(dest2[:, 1].reshape(L // TS, TS, 1),
                             (L // TS, TS, 128))

    wpb = exp_wp.astype(BF)

    ys = pl.pallas_call(
        _routed_block_kernel,
        grid_spec=pltpu.PrefetchScalarGridSpec(
            num_scalar_prefetch=1,
            grid=(HB, NB),
            in_specs=[
                pl.BlockSpec((1, T, 128), lambda j, b, bexp: (b, 0, 0)),
                pl.BlockSpec((1, T, 128), lambda j, b, bexp: (b, 0, 0)),
                pl.BlockSpec((L, DIM), lambda j, b, bexp: (0, 0)),
                pl.BlockSpec((1, DIM, HC), lambda j, b, bexp: (bexp[b], 0, j)),
                pl.BlockSpec((1, DIM, HC), lambda j, b, bexp: (bexp[b], 0, j + HB)),
                pl.BlockSpec((1, HC, DIM), lambda j, b, bexp: (bexp[b], j, 0)),
                pl.BlockSpec((1, DIM, DIM), lambda j, b, bexp: (bexp[b], 0, 0)),
            ],
            out_specs=pl.BlockSpec(
                (T, DIM),
                lambda j, b, bexp: (jnp.where(j == HB - 1, b, 0), 0)),
            scratch_shapes=[pltpu.VMEM((NB, T, DIM), BF)],
        ),
        out_shape=jax.ShapeDtypeStruct((NP, DIM), BF),
    )(bexp, rows_bc, gv_bc, xb, exp_w1, exp_w1, exp_w2, wpb)

    out = pl.pallas_call(
        _combine_shared_kernel,
        grid=(L // TS,),
        in_specs=[
            pl.BlockSpec((TS, DIM), lambda i: (i, 0)),
            pl.BlockSpec((NSH, DIM, 2 * HSH), lambda i: (0, 0, 0)),
            pl.BlockSpec((NSH, HSH, DIM), lambda i: (0, 0, 0)),
            pl.BlockSpec((NP, DIM), lambda i: (0, 0)),
            pl.BlockSpec((1, TS, 128), lambda i: (i, 0, 0)),
            pl.BlockSpec((1, TS, 128), lambda i: (i, 0, 0)),
        ],
        out_specs=pl.BlockSpec((TS, DIM), lambda i: (i, 0)),
        out_shape=jax.ShapeDtypeStruct((L, DIM), jnp.float32),
    )(xb, sh_w1.astype(BF), sh_w2.astype(BF), ys, d0_bc, d1_bc)

    return out.reshape(1, L, DIM), lb_loss


# speedup vs baseline: 1.0822x; 1.0617x over previous
import jax
import jax.numpy as jnp
from jax import lax
from jax.experimental import pallas as pl
from jax.experimental.pallas import tpu as pltpu

DIM = 768
E = 8
TOPK = 2
NSH = 2
L = 2048
H = 4 * DIM
HSH = 2 * DIM

T = 256
NB = 24
NP = NB * T

HB = 2
HC = H // HB

TS = 256
CC = 2048

BF = jnp.bfloat16


def _routed_block_kernel(bexp_ref, rows_ref, gv_ref, x_ref, w1a_ref, w1g_ref,
                         w2_ref, wp_ref, ys_ref, y_s):
    j = pl.program_id(0)
    b = pl.program_id(1)

    rows = rows_ref[0][:, 0:1]
    iota = lax.broadcasted_iota(jnp.int32, (T, L), 1)
    oh = (rows == iota).astype(BF)
    xg = jnp.dot(oh, x_ref[...],
                 preferred_element_type=jnp.float32).astype(BF)

    a = jnp.dot(xg, w1a_ref[0].astype(BF), preferred_element_type=jnp.float32)
    g = jnp.dot(xg, w1g_ref[0].astype(BF), preferred_element_type=jnp.float32)
    u = (a * jax.nn.sigmoid(a) * g).astype(BF)

    part = jnp.dot(u, w2_ref[0].astype(BF), preferred_element_type=jnp.float32)

    @pl.when(j == 0)
    def _():
        y_s[b] = part.astype(BF)

    @pl.when((j > 0) & (j < HB - 1))
    def _():
        y_s[b] = (y_s[b].astype(jnp.float32) + part).astype(BF)

    @pl.when(j == HB - 1)
    def _():
        y = y_s[b].astype(jnp.float32) + part
        z = jnp.dot(y.astype(BF), wp_ref[0], preferred_element_type=jnp.float32)
        gv = gv_ref[0][:, 0:1]
        ys_ref[...] = (z * gv).astype(BF)


def _combine_shared_kernel(x_ref, w1_ref, w2_ref, ys_ref, d0_ref, d1_ref,
                           out_ref):
    d0 = d0_ref[0][:, 0:1]
    d1 = d1_ref[0][:, 0:1]

    acc = jnp.zeros((TS, DIM), jnp.float32)
    for c in range(NP // CC):
        iota = lax.broadcasted_iota(jnp.int32, (TS, CC), 1) + c * CC
        oh2 = (d0 == iota).astype(BF) + (d1 == iota).astype(BF)
        acc = acc + jnp.dot(oh2, ys_ref[c * CC:(c + 1) * CC, :],
                            preferred_element_type=jnp.float32)

    xt = x_ref[...]
    for s in range(NSH):
        h = jnp.dot(xt, w1_ref[s], preferred_element_type=jnp.float32)
        a = h[:, :HSH]
        g = h[:, HSH:]
        u = (a * jax.nn.sigmoid(a) * g).astype(BF)
        y = jnp.dot(u, w2_ref[s], preferred_element_type=jnp.float32)
        acc = acc + (1.0 / NSH) * y
    out_ref[...] = acc


def _bookkeeping_kernel(e_ref, g_ref, rows_ref, gv_ref, bexp_ref, dest_ref):
    e_row = e_ref[...]
    g_row = g_ref[...]

    sub8 = lax.broadcasted_iota(jnp.int32, (E, TOPK * L), 0)
    onehot = (sub8 == e_row).astype(jnp.float32)

    c = onehot
    k = 1
    while k < TOPK * L:
        c = c + jnp.concatenate(
            [jnp.zeros((E, k), jnp.float32), c[:, :-k]], axis=1)
        k *= 2
    rank = ((c - onehot) * onehot).sum(axis=0, keepdims=True)

    ones_row = jnp.ones((1, TOPK * L), jnp.float32)
    counts_row = lax.dot_general(
        ones_row, onehot, (((1,), (1,)), ((), ())),
        precision=lax.Precision.HIGHEST, preferred_element_type=jnp.float32)
    blocks_pe = jnp.floor((counts_row + T - 1) / T)
    bc = blocks_pe
    k = 1
    while k < E:
        bc = bc + jnp.concatenate(
            [jnp.zeros((1, k), jnp.float32), bc[:, :-k]], axis=1)
        k *= 2
    bfirst = bc - blocks_pe
    bfirst_pa = lax.dot_general(
        bfirst, onehot, (((1,), (0,)), ((), ())),
        precision=lax.Precision.HIGHEST, preferred_element_type=jnp.float32)
    dest_f = bfirst_pa * T + rank
    dest_ref[...] = (dest_f + 0.5).astype(jnp.int32)

    iota_nb = lax.broadcasted_iota(jnp.int32, (NB, E), 0).astype(jnp.float32)
    bexp = jnp.sum((iota_nb >= jnp.broadcast_to(bc, (NB, E))).astype(jnp.float32),
                   axis=1, keepdims=True)
    bexp_ref[...] = jnp.minimum(bexp + 0.5, E - 1).astype(jnp.int32)

    tok_row = lax.broadcasted_iota(jnp.int32, (1, TOPK * L), 1) // TOPK
    hi = (tok_row // 256).astype(BF)
    lo = (tok_row % 256).astype(BF)
    g1 = g_row.astype(BF)
    g2 = (g_row - g1.astype(jnp.float32)).astype(BF)
    payload = jnp.concatenate([hi, lo, g1, g2], axis=0)
    desti = (dest_f + 0.5).astype(jnp.int32)
    CB = 1024
    for ci in range(NP // CB):
        sub_p = lax.broadcasted_iota(jnp.int32, (CB, TOPK * L), 0) + ci * CB
        oh = (sub_p == desti).astype(BF)
        res = lax.dot_general(oh, payload, (((1,), (1,)), ((), ())),
                              preferred_element_type=jnp.float32)
        rows_c = (res[:, 0:1] * 256.0 + res[:, 1:2] + 0.5).astype(jnp.int32)
        gv_c = res[:, 2:3] + res[:, 3:4]
        nb0 = ci * (CB // T)
        rows_ref[nb0:nb0 + CB // T] = jnp.broadcast_to(
            rows_c, (CB, 128)).reshape(CB // T, T, 128)
        gv_ref[nb0:nb0 + CB // T] = jnp.broadcast_to(
            gv_c, (CB, 128)).reshape(CB // T, T, 128)


@jax.jit
def kernel(x, gate_w, exp_w1, exp_w2, exp_wp, sh_w1, sh_w2):
    xf = x[0]
    xb = xf.astype(BF)

    gate_logits = x @ gate_w
    top_k_logits, top_k_idx = lax.top_k(gate_logits, TOPK)
    top_k_gates = jax.nn.softmax(top_k_logits, axis=-1)

    expert_mask = jax.nn.one_hot(top_k_idx[:, :, 0], E, dtype=jnp.float32)
    counts0 = expert_mask.sum(axis=(0, 1))
    counts0 = counts0 / counts0.sum()
    target = jnp.ones_like(counts0) / E
    lb_loss = jnp.mean((counts0 - target) ** 2)

    e_row = top_k_idx.reshape(1, TOPK * L)
    g_row = top_k_gates.reshape(1, TOPK * L).astype(jnp.float32)

    rows_bc, gv_bc, bexp2, dest = pl.pallas_call(
        _bookkeeping_kernel,
        in_specs=[
            pl.BlockSpec((1, TOPK * L), lambda: (0, 0)),
            pl.BlockSpec((1, TOPK * L), lambda: (0, 0)),
        ],
        out_specs=[
            pl.BlockSpec((NB, T, 128), lambda: (0, 0, 0)),
            pl.BlockSpec((NB, T, 128), lambda: (0, 0, 0)),
            pl.BlockSpec((NB, 1), lambda: (0, 0)),
            pl.BlockSpec((1, TOPK * L), lambda: (0, 0)),
        ],
        out_shape=[
            jax.ShapeDtypeStruct((NB, T, 128), jnp.int32),
            jax.ShapeDtypeStruct((NB, T, 128), jnp.float32),
            jax.ShapeDtypeStruct((NB, 1), jnp.int32),
            jax.ShapeDtypeStruct((1, TOPK * L), jnp.int32),
        ],
    )(e_row, g_row)
    bexp = bexp2.reshape(NB)

    dest2 = dest.reshape(L, TOPK)
    d0_bc = jnp.broadcast_to(dest2[:, 0].reshape(L // TS, TS, 1),
                             (L // TS, TS, 128))
    d1_bc = jnp.broadcast_to(dest2[:, 1].reshape(L // TS, TS, 1),
                             (L // TS, TS, 128))

    wpb = exp_wp.astype(BF)

    ys = pl.pallas_call(
        _routed_block_kernel,
        grid_spec=pltpu.PrefetchScalarGridSpec(
            num_scalar_prefetch=1,
            grid=(HB, NB),
            in_specs=[
                pl.BlockSpec((1, T, 128), lambda j, b, bexp: (b, 0, 0)),
                pl.BlockSpec((1, T, 128), lambda j, b, bexp: (b, 0, 0)),
                pl.BlockSpec((L, DIM), lambda j, b, bexp: (0, 0)),
                pl.BlockSpec((1, DIM, HC), lambda j, b, bexp: (bexp[b], 0, j)),
                pl.BlockSpec((1, DIM, HC), lambda j, b, bexp: (bexp[b], 0, j + HB)),
                pl.BlockSpec((1, HC, DIM), lambda j, b, bexp: (bexp[b], j, 0)),
                pl.BlockSpec((1, DIM, DIM), lambda j, b, bexp: (bexp[b], 0, 0)),
            ],
            out_specs=pl.BlockSpec(
                (T, DIM),
                lambda j, b, bexp: (jnp.where(j == HB - 1, b, 0), 0)),
            scratch_shapes=[pltpu.VMEM((NB, T, DIM), BF)],
        ),
        out_shape=jax.ShapeDtypeStruct((NP, DIM), BF),
    )(bexp, rows_bc, gv_bc, xb, exp_w1, exp_w1, exp_w2, wpb)

    out = pl.pallas_call(
        _combine_shared_kernel,
        grid=(L // TS,),
        in_specs=[
            pl.BlockSpec((TS, DIM), lambda i: (i, 0)),
            pl.BlockSpec((NSH, DIM, 2 * HSH), lambda i: (0, 0, 0)),
            pl.BlockSpec((NSH, HSH, DIM), lambda i: (0, 0, 0)),
            pl.BlockSpec((NP, DIM), lambda i: (0, 0)),
            pl.BlockSpec((1, TS, 128), lambda i: (i, 0, 0)),
            pl.BlockSpec((1, TS, 128), lambda i: (i, 0, 0)),
        ],
        out_specs=pl.BlockSpec((TS, DIM), lambda i: (i, 0)),
        out_shape=jax.ShapeDtypeStruct((L, DIM), jnp.float32),
    )(xb, sh_w1.astype(BF), sh_w2.astype(BF), ys, d0_bc, d1_bc)

    return out.reshape(1, L, DIM), lb_loss
